# baseline (speedup 1.0000x reference)
import functools

import jax
import jax.numpy as jnp
from jax import lax
from jax.experimental import pallas as pl
from jax.experimental.pallas import tpu as pltpu
from jax.experimental.pallas import tpu_sc as plsc

VOCAB = 1_000_000
EMB_DIM = 64
TOTAL_TOKENS = 819_200

NUM_CORES = 2
NUM_SUBCORES = 16
NUM_WORKERS = NUM_CORES * NUM_SUBCORES
BPW = TOTAL_TOKENS // NUM_WORKERS
CHUNK = 512
NCHUNKS = BPW // CHUNK


def _build_gather():
    mesh = plsc.VectorSubcoreMesh(core_axis_name="c", subcore_axis_name="s")

    @functools.partial(
        pl.kernel,
        mesh=mesh,
        out_type=jax.ShapeDtypeStruct((TOTAL_TOKENS, EMB_DIM), jnp.float32),
        scratch_types=[
            pltpu.VMEM((BPW,), jnp.int32),
            pltpu.VMEM((2, CHUNK, EMB_DIM), jnp.float32),
            pltpu.SemaphoreType.DMA,
            pltpu.SemaphoreType.DMA,
        ],
        compiler_params=pltpu.CompilerParams(use_tc_tiling_on_sc=False),
    )
    def emb_gather(table_hbm, idx_hbm, out_hbm, idx_v, rows_v, gsem, osem):
        wid = lax.axis_index("s") * NUM_CORES + lax.axis_index("c")
        base = wid * BPW
        pltpu.sync_copy(idx_hbm.at[pl.ds(base, BPW)], idx_v)

        def gather(c, buf):
            return pltpu.async_copy(
                table_hbm.at[idx_v.at[pl.ds(c * CHUNK, CHUNK)]],
                rows_v.at[buf],
                gsem,
            )

        gather(0, 0)

        def body(c, carry):
            buf = lax.rem(c, 2)
            pltpu.make_async_copy(
                table_hbm.at[idx_v.at[pl.ds(c * CHUNK, CHUNK)]],
                rows_v.at[buf],
                gsem,
            ).wait()
            @pl.when(c >= 1)
            def _():
                pltpu.make_async_copy(
                    rows_v.at[1 - buf],
                    out_hbm.at[pl.ds(base + (c - 1) * CHUNK, CHUNK)],
                    osem,
                ).wait()

            @pl.when(c + 1 < NCHUNKS)
            def _():
                gather(c + 1, 1 - buf)

            pltpu.async_copy(
                rows_v.at[buf],
                out_hbm.at[pl.ds(base + c * CHUNK, CHUNK)],
                osem,
            )
            return carry

        lax.fori_loop(0, NCHUNKS, body, 0)
        pltpu.make_async_copy(
            rows_v.at[(NCHUNKS - 1) % 2],
            out_hbm.at[pl.ds(base + (NCHUNKS - 1) * CHUNK, CHUNK)],
            osem,
        ).wait()

    return emb_gather


_emb_gather = _build_gather()


def kernel(input, batch_sizes, table):
    emb = _emb_gather(table, input)
    return (emb, batch_sizes)
